# baseline (speedup 1.0000x reference)
import functools

import jax
import jax.numpy as jnp
from jax import lax
from jax.experimental import pallas as pl
from jax.experimental.pallas import tpu as pltpu
from jax.experimental.pallas import tpu_sc as plsc

NC = 2
NS = 16
NW = NC * NS
CSZ = 80
DE = 16


def _perm128():
    p = jnp.arange(128)
    return jnp.eye(128, dtype=jnp.float32)[8 * (p % 16) + p // 16]


def _pack_tile(pv):
    return jnp.concatenate([pv[16 * k:16 * (k + 1), :] for k in range(8)],
                           axis=1)


def _unpack(v8):
    r = v8.shape[0]
    t = jnp.stack([v8[:, 16 * k:16 * (k + 1)] for k in range(8)], axis=1)
    return t.reshape(8 * r, 16)


def _acc_pad(n_nodes):
    rpt = -(-(n_nodes // NS) // 8) * 8
    return rpt, rpt * NS


def _sc_edge_kernel(n_nodes, n_edges, interpret=False):
    nch = n_edges // (NW * CSZ)
    epw = nch * CSZ
    wrows = CSZ * DE // 128
    rpt, n_pad = _acc_pad(n_nodes)

    mesh = plsc.VectorSubcoreMesh(core_axis_name="c", subcore_axis_name="s",
                                  num_cores=NC, num_subcores=NS)

    @functools.partial(
        pl.kernel,
        out_type=(
            jax.ShapeDtypeStruct((n_edges * DE // 128, 128), jnp.float32),
            jax.ShapeDtypeStruct((NC * n_pad, DE), jnp.float32),
            jax.ShapeDtypeStruct((NC * n_pad, DE), jnp.float32),
        ),
        mesh=mesh,
        scratch_types=[
            pltpu.VMEM((nch, CSZ), jnp.int32),
            pltpu.VMEM((nch, CSZ), jnp.int32),
            pltpu.VMEM((CSZ, DE), jnp.float32),
            pltpu.VMEM((CSZ, DE), jnp.float32),
            pltpu.VMEM((CSZ, DE), jnp.float32),
            pltpu.VMEM((CSZ, DE), jnp.float32),
            pltpu.VMEM((wrows, 128), jnp.float32),
            pltpu.VMEM((CSZ, DE), jnp.float32),
            pltpu.VMEM((rpt, DE), jnp.float32),
            pltpu.VMEM_SHARED((n_pad, DE), jnp.float32),
            pltpu.VMEM_SHARED((n_pad, DE), jnp.float32),
            pltpu.SemaphoreType.DMA,
            pltpu.SemaphoreType.DMA,
        ],
        compiler_params=pltpu.CompilerParams(use_tc_tiling_on_sc=False),
        interpret=interpret,
    )
    def k(a_hbm, b_hbm, c_hbm, src_hbm, dst_hbm, ones_hbm, zeros_hbm,
          he_hbm, sums_hbm, cnts_hbm,
          srcv, dstv, a_v, b_v, c_v, he_v, hew_v, ones_v, z_v,
          sums_sh, cnts_sh, sem_a, sem_b):
        cid = lax.axis_index("c")
        sid = lax.axis_index("s")
        wid = sid * NC + cid

        pltpu.sync_copy(ones_hbm, ones_v)
        pltpu.sync_copy(zeros_hbm, z_v)
        pltpu.sync_copy(src_hbm.at[wid], srcv)
        pltpu.sync_copy(dst_hbm.at[wid], dstv)
        pltpu.sync_copy(z_v, sums_sh.at[pl.ds(sid * rpt, rpt)])
        pltpu.sync_copy(z_v, cnts_sh.at[pl.ds(sid * rpt, rpt)])
        plsc.subcore_barrier()

        def chunk(j, carry):
            base = wid * epw + j * CSZ
            ga = pltpu.async_copy(a_hbm.at[srcv.at[j]], a_v, sem_a)
            gb = pltpu.async_copy(b_hbm.at[dstv.at[j]], b_v, sem_b)
            pltpu.sync_copy(c_hbm.at[pl.ds(base, CSZ)], c_v)
            ga.wait()
            gb.wait()

            def row(r, c2):
                for kk in range(8):
                    i = 8 * r + kk
                    h = a_v[i, :] + b_v[i, :] + c_v[i, :]
                    he_v[i, :] = h
                    hew_v[r, 16 * kk:16 * (kk + 1)] = h
                return c2

            lax.fori_loop(0, wrows, row, 0)
            pltpu.sync_copy(hew_v,
                            he_hbm.at[pl.ds(base * DE // 128, wrows)])
            pltpu.sync_copy(he_v, sums_sh.at[dstv.at[j]], add=True)
            pltpu.sync_copy(ones_v, cnts_sh.at[dstv.at[j]], add=True)
            return carry

        lax.fori_loop(0, nch, chunk, 0)

        plsc.subcore_barrier()
        out0 = cid * n_pad + sid * rpt
        pltpu.sync_copy(sums_sh.at[pl.ds(sid * rpt, rpt)],
                        sums_hbm.at[pl.ds(out0, rpt)])
        pltpu.sync_copy(cnts_sh.at[pl.ds(sid * rpt, rpt)],
                        cnts_hbm.at[pl.ds(out0, rpt)])

    return k


def _tc_ab(x, w12, S, n_tbl):
    n = x.shape[0]
    nsub = n_tbl // 128

    def body(x_ref, w_ref, S_ref, oa_ref, ob_ref):
        r = jnp.dot(x_ref[...], w_ref[...], preferred_element_type=jnp.float32)
        r = jnp.concatenate(
            [r, jnp.zeros((n_tbl - n, 2 * DE), jnp.float32)], axis=0)
        Sm = S_ref[...]
        for s in range(nsub):
            pr = jnp.dot(Sm, r[128 * s:128 * (s + 1), :],
                         preferred_element_type=jnp.float32,
                         precision=jax.lax.Precision.HIGHEST)
            oa_ref[16 * s:16 * (s + 1), :] = _pack_tile(pr[:, :DE])
            ob_ref[16 * s:16 * (s + 1), :] = _pack_tile(pr[:, DE:])

    return pl.pallas_call(
        body,
        out_shape=(
            jax.ShapeDtypeStruct((n_tbl // 8, 128), jnp.float32),
            jax.ShapeDtypeStruct((n_tbl // 8, 128), jnp.float32),
        ),
    )(x, w12, S)


def _tc_c(eaT, w3, b_e, S):
    n_edges = eaT.shape[1]
    grid = 10
    blk = n_edges // grid
    nsub = blk // 128

    def body(ea_ref, w_ref, b_ref, S_ref, o_ref):
        Sm = S_ref[...]
        ea = ea_ref[...]
        w = w_ref[...]
        bias = b_ref[...]
        for t in range(nsub):
            pv = jax.lax.dot_general(
                Sm, ea[:, 128 * t:128 * (t + 1)],
                (((1,), (1,)), ((), ())),
                preferred_element_type=jnp.float32,
                precision=jax.lax.Precision.HIGHEST)
            pc = jnp.dot(pv, w, preferred_element_type=jnp.float32) + bias
            o_ref[16 * t:16 * (t + 1), :] = _pack_tile(pc)

    return pl.pallas_call(
        body,
        grid=(grid,),
        in_specs=[
            pl.BlockSpec((DE, blk), lambda i: (0, i)),
            pl.BlockSpec((DE, DE), lambda i: (0, 0)),
            pl.BlockSpec((1, DE), lambda i: (0, 0)),
            pl.BlockSpec((128, 128), lambda i: (0, 0)),
        ],
        out_specs=pl.BlockSpec((blk // 8, 128), lambda i: (i, 0)),
        out_shape=jax.ShapeDtypeStruct((n_edges // 8, 128), jnp.float32),
    )(eaT, w3, b_e, S)


def _tc_post(x, sums8, cnts8, wn_x, wn_a, b_n, n_pad):
    n = x.shape[0]
    rows = n * DE // 128
    rowsp = n_pad * DE // 128

    def body(x_ref, s_ref, c_ref, wx_ref, wa_ref, b_ref, o_ref):
        sums = s_ref[pl.ds(0, rows)] + s_ref[pl.ds(rowsp, rows)]
        cnts = jnp.maximum(c_ref[pl.ds(0, rows)] + c_ref[pl.ds(rowsp, rows)],
                           1.0)
        agg = _unpack(sums / cnts)
        o_ref[...] = (
            jnp.dot(x_ref[...], wx_ref[...], preferred_element_type=jnp.float32)
            + jnp.dot(agg, wa_ref[...], preferred_element_type=jnp.float32)
            + b_ref[...]
        )

    return pl.pallas_call(
        body,
        out_shape=jax.ShapeDtypeStruct((x.shape[0], x.shape[1]), jnp.float32),
    )(x, sums8, cnts8, wn_x, wn_a, b_n)


def kernel(x, edge_index, edge_attr, W_edge, b_edge, W_node, b_node):
    n_nodes, d_node = x.shape
    n_edges = edge_index.shape[1]
    n_tbl = -(-n_nodes // 128) * 128

    w12 = jnp.concatenate([W_edge[:d_node], W_edge[d_node:2 * d_node]],
                          axis=1)
    w3 = W_edge[2 * d_node:]
    S = _perm128()

    a8, b8 = _tc_ab(x, w12, S, n_tbl)
    c8 = _tc_c(edge_attr.T, w3, b_edge.reshape(1, DE), S)

    a = a8.reshape(n_tbl, DE)
    b = b8.reshape(n_tbl, DE)
    c = c8.reshape(n_edges, DE)

    nch = n_edges // (NW * CSZ)
    rpt, n_pad = _acc_pad(n_nodes)
    src = edge_index[0].reshape(NW, nch, CSZ)
    dst = edge_index[1].reshape(NW, nch, CSZ)
    ones = jnp.ones((CSZ, DE), jnp.float32)
    zeros = jnp.zeros((rpt, DE), jnp.float32)

    he8, sums_p, cnts_p = _sc_edge_kernel(n_nodes, n_edges)(
        a, b, c, src, dst, ones, zeros)

    h_v = _tc_post(
        x,
        sums_p.reshape(NC * n_pad * DE // 128, 128),
        cnts_p.reshape(NC * n_pad * DE // 128, 128),
        W_node[:d_node],
        W_node[d_node:],
        b_node.reshape(1, d_node),
        n_pad,
    )
    h_e = he8.reshape(n_edges, DE)
    return (h_v, edge_index, h_e)


# speedup vs baseline: 2.7412x; 2.7412x over previous
import functools

import jax
import jax.numpy as jnp
from jax import lax
from jax.experimental import pallas as pl
from jax.experimental.pallas import tpu as pltpu
from jax.experimental.pallas import tpu_sc as plsc

NC = 2
NS = 16
NW = NC * NS
CSZ = 80
DE = 16


def _perm128():
    p = jnp.arange(128)
    return jnp.eye(128, dtype=jnp.float32)[8 * (p % 16) + p // 16]


def _pack_tile(pv):
    return jnp.concatenate([pv[16 * k:16 * (k + 1), :] for k in range(8)],
                           axis=1)


def _unpack(v8):
    r = v8.shape[0]
    t = jnp.stack([v8[:, 16 * k:16 * (k + 1)] for k in range(8)], axis=1)
    return t.reshape(8 * r, 16)


def _acc_pad(n_nodes):
    rpt = -(-(n_nodes // NS) // 8) * 8
    return rpt, rpt * NS


def _sc_edge_kernel(n_nodes, n_edges, interpret=False):
    nch = n_edges // (NW * CSZ)
    epw = nch * CSZ
    wrows = CSZ * DE // 128
    rpt, n_pad = _acc_pad(n_nodes)

    mesh = plsc.VectorSubcoreMesh(core_axis_name="c", subcore_axis_name="s",
                                  num_cores=NC, num_subcores=NS)

    @functools.partial(
        pl.kernel,
        out_type=(
            jax.ShapeDtypeStruct((n_edges * DE // 128, 128), jnp.float32),
            jax.ShapeDtypeStruct((NC * n_pad, DE), jnp.float32),
            jax.ShapeDtypeStruct((NC * n_pad, DE), jnp.float32),
        ),
        mesh=mesh,
        scratch_types=[
            pltpu.VMEM((nch, CSZ), jnp.int32),
            pltpu.VMEM((nch, CSZ), jnp.int32),
            pltpu.VMEM((2, CSZ, DE), jnp.float32),
            pltpu.VMEM((2, CSZ, DE), jnp.float32),
            pltpu.VMEM((2, CSZ, DE), jnp.float32),
            pltpu.VMEM((2, CSZ, DE), jnp.float32),
            pltpu.VMEM((2, wrows, 128), jnp.float32),
            pltpu.VMEM((CSZ, DE), jnp.float32),
            pltpu.VMEM((rpt, DE), jnp.float32),
            pltpu.VMEM_SHARED((n_pad, DE), jnp.float32),
            pltpu.VMEM_SHARED((n_pad, DE), jnp.float32),
            pltpu.SemaphoreType.DMA,
            pltpu.SemaphoreType.DMA,
            pltpu.SemaphoreType.DMA,
            pltpu.SemaphoreType.DMA,
            pltpu.SemaphoreType.DMA,
            pltpu.SemaphoreType.DMA,
            pltpu.SemaphoreType.DMA,
            pltpu.SemaphoreType.DMA,
        ],
        compiler_params=pltpu.CompilerParams(use_tc_tiling_on_sc=False),
        interpret=interpret,
    )
    def k(a_hbm, b_hbm, c_hbm, src_hbm, dst_hbm, ones_hbm, zeros_hbm,
          he_hbm, sums_hbm, cnts_hbm,
          srcv, dstv, a_v, b_v, c_v, he_v, hew_v, ones_v, z_v,
          sums_sh, cnts_sh,
          sem_a0, sem_a1, sem_b0, sem_b1, sem_c0, sem_c1, sem_s0, sem_s1):
        cid = lax.axis_index("c")
        sid = lax.axis_index("s")
        wid = sid * NC + cid

        pltpu.sync_copy(ones_hbm, ones_v)
        pltpu.sync_copy(zeros_hbm, z_v)
        pltpu.sync_copy(src_hbm.at[wid], srcv)
        pltpu.sync_copy(dst_hbm.at[wid], dstv)
        pltpu.sync_copy(z_v, sums_sh.at[pl.ds(sid * rpt, rpt)])
        pltpu.sync_copy(z_v, cnts_sh.at[pl.ds(sid * rpt, rpt)])
        plsc.subcore_barrier()

        def issue(j, p, sa, sb, sc):
            base = wid * epw + j * CSZ
            pltpu.async_copy(a_hbm.at[srcv.at[j]], a_v.at[p], sa)
            pltpu.async_copy(b_hbm.at[dstv.at[j]], b_v.at[p], sb)
            pltpu.async_copy(c_hbm.at[pl.ds(base, CSZ)], c_v.at[p], sc)

        def process(j, p, sa, sb, sc):
            base = wid * epw + j * CSZ
            pltpu.make_async_copy(a_hbm.at[srcv.at[j]], a_v.at[p], sa).wait()
            pltpu.make_async_copy(b_hbm.at[dstv.at[j]], b_v.at[p], sb).wait()
            pltpu.make_async_copy(c_hbm.at[pl.ds(base, CSZ)], c_v.at[p],
                                  sc).wait()

            def row(r, c2):
                for kk in range(8):
                    i = 8 * r + kk
                    h = a_v[p, i, :] + b_v[p, i, :] + c_v[p, i, :]
                    he_v[p, i, :] = h
                    hew_v[p, r, 16 * kk:16 * (kk + 1)] = h
                return c2

            lax.fori_loop(0, wrows, row, 0)
            pltpu.sync_copy(hew_v.at[p],
                            he_hbm.at[pl.ds(base * DE // 128, wrows)])
            pltpu.sync_copy(he_v.at[p], sums_sh.at[dstv.at[j]], add=True)
            pltpu.sync_copy(ones_v, cnts_sh.at[dstv.at[j]], add=True)

        issue(0, 0, sem_a0, sem_b0, sem_c0)

        def body(j2, carry):
            j = 2 * j2
            issue(j + 1, 1, sem_a1, sem_b1, sem_c1)
            process(j, 0, sem_a0, sem_b0, sem_c0)
            issue(j + 2, 0, sem_a0, sem_b0, sem_c0)
            process(j + 1, 1, sem_a1, sem_b1, sem_c1)
            return carry

        lax.fori_loop(0, (nch - 1) // 2, body, 0)

        process(nch - 1, 0, sem_a0, sem_b0, sem_c0)

        plsc.subcore_barrier()
        out0 = cid * n_pad + sid * rpt
        pltpu.sync_copy(sums_sh.at[pl.ds(sid * rpt, rpt)],
                        sums_hbm.at[pl.ds(out0, rpt)])
        pltpu.sync_copy(cnts_sh.at[pl.ds(sid * rpt, rpt)],
                        cnts_hbm.at[pl.ds(out0, rpt)])

    return k


def _tc_ab(x, w12, S, n_tbl):
    n = x.shape[0]
    nsub = n_tbl // 128

    def body(x_ref, w_ref, S_ref, oa_ref, ob_ref):
        r = jnp.dot(x_ref[...], w_ref[...], preferred_element_type=jnp.float32)
        r = jnp.concatenate(
            [r, jnp.zeros((n_tbl - n, 2 * DE), jnp.float32)], axis=0)
        Sm = S_ref[...]
        for s in range(nsub):
            pr = jnp.dot(Sm, r[128 * s:128 * (s + 1), :],
                         preferred_element_type=jnp.float32)
            oa_ref[16 * s:16 * (s + 1), :] = _pack_tile(pr[:, :DE])
            ob_ref[16 * s:16 * (s + 1), :] = _pack_tile(pr[:, DE:])

    return pl.pallas_call(
        body,
        out_shape=(
            jax.ShapeDtypeStruct((n_tbl // 8, 128), jnp.float32),
            jax.ShapeDtypeStruct((n_tbl // 8, 128), jnp.float32),
        ),
    )(x, w12, S)


def _tc_c(eaT, w3, b_e, S):
    n_edges = eaT.shape[1]
    grid = 10
    blk = n_edges // grid
    nsub = blk // 128

    def body(ea_ref, w_ref, b_ref, S_ref, o_ref):
        Sm = S_ref[...]
        ea = ea_ref[...]
        w = w_ref[...]
        bias = b_ref[...]
        for t in range(nsub):
            pv = jax.lax.dot_general(
                Sm, ea[:, 128 * t:128 * (t + 1)],
                (((1,), (1,)), ((), ())),
                preferred_element_type=jnp.float32)
            pc = jnp.dot(pv, w, preferred_element_type=jnp.float32) + bias
            o_ref[16 * t:16 * (t + 1), :] = _pack_tile(pc)

    return pl.pallas_call(
        body,
        grid=(grid,),
        in_specs=[
            pl.BlockSpec((DE, blk), lambda i: (0, i)),
            pl.BlockSpec((DE, DE), lambda i: (0, 0)),
            pl.BlockSpec((1, DE), lambda i: (0, 0)),
            pl.BlockSpec((128, 128), lambda i: (0, 0)),
        ],
        out_specs=pl.BlockSpec((blk // 8, 128), lambda i: (i, 0)),
        out_shape=jax.ShapeDtypeStruct((n_edges // 8, 128), jnp.float32),
    )(eaT, w3, b_e, S)


def _tc_post(x, sums8, cnts8, wn_x, wn_a, b_n, n_pad):
    n = x.shape[0]
    rows = n * DE // 128
    rowsp = n_pad * DE // 128

    def body(x_ref, s_ref, c_ref, wx_ref, wa_ref, b_ref, o_ref):
        sums = s_ref[pl.ds(0, rows)] + s_ref[pl.ds(rowsp, rows)]
        cnts = jnp.maximum(c_ref[pl.ds(0, rows)] + c_ref[pl.ds(rowsp, rows)],
                           1.0)
        agg = _unpack(sums / cnts)
        o_ref[...] = (
            jnp.dot(x_ref[...], wx_ref[...], preferred_element_type=jnp.float32)
            + jnp.dot(agg, wa_ref[...], preferred_element_type=jnp.float32)
            + b_ref[...]
        )

    return pl.pallas_call(
        body,
        out_shape=jax.ShapeDtypeStruct((x.shape[0], x.shape[1]), jnp.float32),
    )(x, sums8, cnts8, wn_x, wn_a, b_n)


def kernel(x, edge_index, edge_attr, W_edge, b_edge, W_node, b_node):
    n_nodes, d_node = x.shape
    n_edges = edge_index.shape[1]
    n_tbl = -(-n_nodes // 128) * 128

    w12 = jnp.concatenate([W_edge[:d_node], W_edge[d_node:2 * d_node]],
                          axis=1)
    w3 = W_edge[2 * d_node:]
    S = _perm128()

    a8, b8 = _tc_ab(x, w12, S, n_tbl)
    c8 = _tc_c(edge_attr.T, w3, b_edge.reshape(1, DE), S)

    a = a8.reshape(n_tbl, DE)
    b = b8.reshape(n_tbl, DE)
    c = c8.reshape(n_edges, DE)

    nch = n_edges // (NW * CSZ)
    rpt, n_pad = _acc_pad(n_nodes)
    src = edge_index[0].reshape(NW, nch, CSZ)
    dst = edge_index[1].reshape(NW, nch, CSZ)
    ones = jnp.ones((CSZ, DE), jnp.float32)
    zeros = jnp.zeros((rpt, DE), jnp.float32)

    he8, sums_p, cnts_p = _sc_edge_kernel(n_nodes, n_edges)(
        a, b, c, src, dst, ones, zeros)

    h_v = _tc_post(
        x,
        sums_p.reshape(NC * n_pad * DE // 128, 128),
        cnts_p.reshape(NC * n_pad * DE // 128, 128),
        W_node[:d_node],
        W_node[d_node:],
        b_node.reshape(1, d_node),
        n_pad,
    )
    h_e = he8.reshape(n_edges, DE)
    return (h_v, edge_index, h_e)
